# baseline (device time: 182524 ns/iter reference)
import jax
import jax.numpy as jnp
from jax import lax
from jax.experimental import pallas as pl
from jax.experimental.pallas import tpu as pltpu

CQ = 32
NSLOTS = 8
NVK = 4
HALF = CQ // 2

DIAGZ = [i for i in range(CQ) if i % 3 == 2]
DIAG_X = [i for i in range(CQ) if i % 3 != 2 and i % 2 == 1]
DIAG_Y = [i for i in range(CQ) if i % 3 != 2 and i % 2 == 0]
_XPOS = {i: j for j, i in enumerate(DIAG_X)}
_YPOS = {i: j for j, i in enumerate(DIAG_Y)}
_ZPOS = {i: j for j, i in enumerate(DIAGZ)}


def _make_plan(cq):
    seq = [("z", 0), ("z", 1)]
    zi = 2
    for i in range(cq):
        seq.append(("xd", i) if i % 2 == 0 else ("yd", i))
        if zi < cq:
            seq.append(("z", zi))
            zi += 1
    casts = [("s", i) for i in range(2, cq)] + [
        ("k", t) for t in range(3 * cq)
    ]
    plan = [([("s", 0), ("s", 1)], None)]
    idx = 0
    for w in seq:
        plan.append((casts[idx:idx + 2], w))
        idx += 2
    assert idx >= len(casts)
    return plan


_PLAN = _make_plan(CQ)


def kernel(x):
    m, n = x.shape
    qh = m // 4
    c = qh // CQ

    def body(x_ref, out_ref, vx, vc, vk, vzq, load_sems, store_sems,
             z_send, z_recv, xd_send, xd_recv, yd_send, yd_recv,
             xg_send, xg_recv, yg_send, yg_recv, zq_send, zq_recv):
        my_x = lax.axis_index("x")
        my_y = lax.axis_index("y")
        my_z = lax.axis_index("z")
        z_peer = (my_x, my_y, 1 - my_z)
        x_peer = (1 - my_x, my_y, my_z)
        y_peer = (my_x, 1 - my_y, my_z)

        barrier_sem = pltpu.get_barrier_semaphore()
        for peer in (z_peer, x_peer, y_peer):
            pl.semaphore_signal(
                barrier_sem, inc=1, device_id=peer,
                device_id_type=pl.DeviceIdType.MESH,
            )
        pl.semaphore_wait(barrier_sem, 3)

        my_base = my_z * m
        oth_base = (1 - my_z) * m
        q_mine = (2 * my_x + my_y) * qh
        q_x = (2 * (1 - my_x) + my_y) * qh
        q_y = (2 * my_x + (1 - my_y)) * qh
        q_d = (2 * (1 - my_x) + (1 - my_y)) * qh

        def rdma(row, send_sem, recv_sem, peer, src=None):
            dst = out_ref.at[pl.ds(row, c), :]
            return pltpu.make_async_remote_copy(
                src_ref=dst if src is None else src,
                dst_ref=dst, send_sem=send_sem, recv_sem=recv_sem,
                device_id=peer, device_id_type=pl.DeviceIdType.MESH,
            )

        xd_in = [rdma(oth_base + q_x + i * c, xd_send.at[i],
                      xd_recv.at[i], x_peer) for i in range(CQ)]
        yd_in = [rdma(oth_base + q_y + i * c, yd_send.at[i],
                      yd_recv.at[i], y_peer) for i in range(CQ)]
        xg_in = [rdma(oth_base + q_d + i * c, xg_send.at[j],
                      xg_recv.at[j], x_peer) for j, i in enumerate(DIAG_X)]
        yg_in = [rdma(oth_base + q_d + i * c, yg_send.at[j],
                      yg_recv.at[j], y_peer) for j, i in enumerate(DIAG_Y)]
        zg_in = [rdma(oth_base + q_d + i * c, zq_send.at[j],
                      zq_recv.at[j], z_peer) for j, i in enumerate(DIAGZ)]

        keep_offs = [q_x, q_y, q_d]
        casts = [ci for cast_items, _ in _PLAN for ci in cast_items]
        lmap = {}
        loads = []
        for li, (kind, idx) in enumerate(casts):
            if kind == "s":
                off = q_mine + idx * c
            else:
                off = keep_offs[idx // CQ] + (idx % CQ) * c
            loads.append(
                pltpu.make_async_copy(
                    x_ref.at[pl.ds(off, c), :], vx.at[li % NSLOTS],
                    load_sems.at[li % NSLOTS],
                )
            )
            lmap[(kind, idx)] = (li, off)

        state = {"started": 0, "casted": 0}

        def pump_loads():
            while (state["started"] < len(loads)
                   and state["started"] - state["casted"] < NSLOTS - 1):
                loads[state["started"]].start()
                state["started"] += 1

        stores = {}
        store_waited = set()
        z_rdmas = [None] * CQ
        fwds = []

        vk_state = {"count": 0, "last": [None] * NVK}

        def do_cast(kind, idx):
            li, off = lmap[(kind, idx)]
            loads[li].wait()
            state["casted"] += 1
            pump_loads()
            diag_i = idx - 2 * CQ if (kind == "k" and idx >= 2 * CQ) else None
            if kind == "s":
                vc[idx] = vx[li % NSLOTS].astype(jnp.bfloat16)
                buf = vc.at[idx]
                sidx = idx
            elif diag_i is not None and diag_i in _ZPOS:
                j = _ZPOS[diag_i]
                vzq[j] = vx[li % NSLOTS].astype(jnp.bfloat16)
                buf = vzq.at[j]
                sidx = CQ + idx
            else:
                slot = vk_state["count"] % NVK
                vk_state["count"] += 1
                prev = vk_state["last"][slot]
                if prev is not None and prev not in store_waited:
                    stores[prev].wait()
                    store_waited.add(prev)
                vk_state["last"][slot] = (kind, idx)
                vk[slot] = vx[li % NSLOTS].astype(jnp.bfloat16)
                buf = vk.at[slot]
                sidx = CQ + idx
            st = pltpu.make_async_copy(
                buf, out_ref.at[pl.ds(my_base + off, c), :],
                store_sems.at[sidx],
            )
            st.start()
            stores[(kind, idx)] = st
            if kind == "s":
                rd = rdma(my_base + off, z_send.at[idx], z_recv.at[idx],
                          z_peer, src=vc.at[idx])
                rd.start()
                z_rdmas[idx] = rd
            elif diag_i is not None and diag_i in _ZPOS:
                fw = rdma(my_base + off, zq_send.at[_ZPOS[diag_i]],
                          zq_recv.at[_ZPOS[diag_i]], z_peer,
                          src=vzq.at[_ZPOS[diag_i]])
                fw.start()
                fwds.append(fw)

        def do_wait(stream, i):
            if stream == "z":
                z_rdmas[i].wait_recv()
                row = oth_base + q_mine + i * c
                for sem_s, sem_r, peer in (
                    (xd_send.at[i], xd_recv.at[i], x_peer),
                    (yd_send.at[i], yd_recv.at[i], y_peer),
                ):
                    fw = rdma(row, sem_s, sem_r, peer)
                    fw.start()
                    fwds.append(fw)
            elif stream == "xd":
                xd_in[i].wait_recv()
                if i in _YPOS:
                    fw = rdma(oth_base + q_x + i * c, yg_send.at[_YPOS[i]],
                              yg_recv.at[_YPOS[i]], y_peer)
                    fw.start()
                    fwds.append(fw)
            else:
                yd_in[i].wait_recv()
                if i in _XPOS:
                    fw = rdma(oth_base + q_y + i * c, xg_send.at[_XPOS[i]],
                              xg_recv.at[_XPOS[i]], x_peer)
                    fw.start()
                    fwds.append(fw)

        pump_loads()
        for cast_items, wait_item in _PLAN:
            for kind, idx in cast_items:
                do_cast(kind, idx)
            if wait_item is not None:
                do_wait(*wait_item)

        for i in range(1, CQ, 2):
            xd_in[i].wait_recv()
        for i in range(0, CQ, 2):
            yd_in[i].wait_recv()
        for d in xg_in + yg_in + zg_in:
            d.wait_recv()
        for key, st in stores.items():
            if key not in store_waited:
                st.wait()
        for rd in z_rdmas:
            rd.wait_send()
        for fw in fwds:
            fw.wait_send()

    return pl.pallas_call(
        body,
        out_shape=jax.ShapeDtypeStruct((2 * m, n), jnp.bfloat16),
        in_specs=[pl.BlockSpec(memory_space=pl.ANY)],
        out_specs=pl.BlockSpec(memory_space=pl.ANY),
        scratch_shapes=[
            pltpu.VMEM((NSLOTS, c, n), jnp.float32),
            pltpu.VMEM((CQ, c, n), jnp.bfloat16),
            pltpu.VMEM((NVK, c, n), jnp.bfloat16),
            pltpu.VMEM((len(DIAGZ), c, n), jnp.bfloat16),
            pltpu.SemaphoreType.DMA((NSLOTS,)),
            pltpu.SemaphoreType.DMA((4 * CQ,)),
            pltpu.SemaphoreType.DMA((CQ,)),
            pltpu.SemaphoreType.DMA((CQ,)),
            pltpu.SemaphoreType.DMA((CQ,)),
            pltpu.SemaphoreType.DMA((CQ,)),
            pltpu.SemaphoreType.DMA((CQ,)),
            pltpu.SemaphoreType.DMA((CQ,)),
            pltpu.SemaphoreType.DMA((len(DIAG_X),)),
            pltpu.SemaphoreType.DMA((len(DIAG_X),)),
            pltpu.SemaphoreType.DMA((len(DIAG_Y),)),
            pltpu.SemaphoreType.DMA((len(DIAG_Y),)),
            pltpu.SemaphoreType.DMA((len(DIAGZ),)),
            pltpu.SemaphoreType.DMA((len(DIAGZ),)),
        ],
        compiler_params=pltpu.CompilerParams(collective_id=0),
    )(x)
